# baseline (speedup 1.0000x reference)
import functools

import jax
import jax.numpy as jnp
from jax import lax
from jax.experimental import pallas as pl
from jax.experimental.pallas import tpu as pltpu
from jax.experimental.pallas import tpu_sc as plsc

_info = plsc.get_sparse_core_info()
_NC, _NS = _info.num_cores, _info.num_subcores
_NW = _NC * _NS

_CHUNK = 6400


@functools.lru_cache(maxsize=None)
def _build(n_idx: int, n_rows: int):
    assert n_idx % (_NW * _CHUNK) == 0
    ch_per_w = n_idx // (_NW * _CHUNK)

    mesh = plsc.VectorSubcoreMesh(core_axis_name="c", subcore_axis_name="s")

    @functools.partial(
        pl.kernel,
        mesh=mesh,
        out_type=jax.ShapeDtypeStruct((_NW, ch_per_w, _CHUNK), jnp.float32),
        scratch_types=[
            pltpu.VMEM((ch_per_w, _CHUNK), jnp.int32),
            pltpu.VMEM((ch_per_w, _CHUNK), jnp.float32),
            pltpu.SemaphoreType.DMA,
        ],
    )
    def gather_kernel(idx_hbm, tab_hbm, out_hbm, idx_v, rows_v, sem):
        wid = lax.axis_index("s") * _NC + lax.axis_index("c")
        pltpu.sync_copy(idx_hbm.at[wid], idx_v)

        def step(j, carry):
            pltpu.async_copy(tab_hbm.at[idx_v.at[j]], rows_v.at[j], sem)
            return carry

        lax.fori_loop(0, ch_per_w, step, 0, unroll=False)
        pltpu.make_async_copy(out_hbm.at[wid], rows_v, sem).wait()
        pltpu.sync_copy(rows_v, out_hbm.at[wid])

    return gather_kernel


def kernel(input, table):
    b, h = input.shape
    n_idx = b * h
    idx3d = input.reshape(_NW, n_idx // (_NW * _CHUNK), _CHUNK)
    tab = table.reshape(-1)
    out = _build(n_idx, tab.shape[0])(idx3d, tab)
    return out.reshape(b, h, 1)


# speedup vs baseline: 1.1490x; 1.1490x over previous
import functools

import jax
import jax.numpy as jnp
from jax import lax
from jax.experimental import pallas as pl
from jax.experimental.pallas import tpu as pltpu
from jax.experimental.pallas import tpu_sc as plsc

_info = plsc.get_sparse_core_info()
_NC, _NS = _info.num_cores, _info.num_subcores
_NW = _NC * _NS

_CHUNK = 6400


@functools.lru_cache(maxsize=None)
def _build(n_idx: int, n_rows: int):
    assert n_idx % (_NW * _CHUNK) == 0
    ch_per_w = n_idx // (_NW * _CHUNK)

    mesh = plsc.VectorSubcoreMesh(core_axis_name="c", subcore_axis_name="s")

    @functools.partial(
        pl.kernel,
        mesh=mesh,
        out_type=jax.ShapeDtypeStruct((_NW, ch_per_w, _CHUNK), jnp.float32),
        scratch_types=[
            pltpu.VMEM((ch_per_w, _CHUNK), jnp.int32),
            pltpu.VMEM((ch_per_w, _CHUNK), jnp.float32),
            pltpu.VMEM_SHARED((n_rows,), jnp.float32),
            pltpu.SemaphoreType.DMA,
        ],
    )
    def gather_kernel(idx_hbm, tab_hbm, out_hbm, idx_v, rows_v, tab_sh, sem):
        wid = lax.axis_index("s") * _NC + lax.axis_index("c")
        @pl.when(lax.axis_index("s") == 0)
        def _stage():
            pltpu.sync_copy(tab_hbm, tab_sh)

        pltpu.sync_copy(idx_hbm.at[wid], idx_v)
        plsc.subcore_barrier()

        def step(j, carry):
            pltpu.async_copy(tab_sh.at[idx_v.at[j]], rows_v.at[j], sem)
            return carry

        lax.fori_loop(0, ch_per_w, step, 0, unroll=False)
        pltpu.make_async_copy(out_hbm.at[wid], rows_v, sem).wait()
        pltpu.sync_copy(rows_v, out_hbm.at[wid])

    return gather_kernel


def kernel(input, table):
    b, h = input.shape
    n_idx = b * h
    idx3d = input.reshape(_NW, n_idx // (_NW * _CHUNK), _CHUNK)
    tab = table.reshape(-1)
    out = _build(n_idx, tab.shape[0])(idx3d, tab)
    return out.reshape(b, h, 1)
